# baseline (speedup 1.0000x reference)
import functools

import jax
import jax.numpy as jnp
from jax import lax
from jax.experimental import pallas as pl
from jax.experimental.pallas import tpu as pltpu
from jax.experimental.pallas import tpu_sc as plsc

NC = 2
NS = 16
NW = NC * NS
CHUNK = 128


def _sc_gather(idx3, table, B, D, n_chunks):
    mesh = plsc.VectorSubcoreMesh(core_axis_name="c", subcore_axis_name="s")
    b_per_w = n_chunks * CHUNK

    @functools.partial(
        pl.kernel,
        mesh=mesh,
        out_type=jax.ShapeDtypeStruct((B, D), jnp.float32),
        compiler_params=pltpu.CompilerParams(use_tc_tiling_on_sc=False),
        scratch_types=[
            pltpu.VMEM((n_chunks, CHUNK), jnp.int32),
            pltpu.VMEM((n_chunks, CHUNK, D), jnp.float32),
            pltpu.SemaphoreType.DMA,
            pltpu.SemaphoreType.DMA,
        ],
    )
    def k(idx_hbm, table_hbm, out_hbm, idx_v, rows_v, gsem, ssem):
        wid = lax.axis_index("s") * NC + lax.axis_index("c")
        base = wid * b_per_w
        pltpu.sync_copy(idx_hbm.at[wid], idx_v)
        gathers = [
            pltpu.async_copy(table_hbm.at[idx_v.at[j]], rows_v.at[j], gsem)
            for j in range(n_chunks)
        ]
        stores = []
        for j in range(n_chunks):
            gathers[j].wait()
            stores.append(
                pltpu.async_copy(
                    rows_v.at[j], out_hbm.at[pl.ds(base + j * CHUNK, CHUNK)], ssem
                )
            )
        for s in stores:
            s.wait()

    return k(idx3, table)


def kernel(drug_ids, table):
    B, = drug_ids.shape
    _, D = table.shape
    n_chunks = B // (NW * CHUNK)
    idx3 = drug_ids.astype(jnp.int32).reshape(NW, n_chunks, CHUNK)
    return _sc_gather(idx3, table, B, D, n_chunks)


# speedup vs baseline: 1.7195x; 1.7195x over previous
import functools

import jax
import jax.numpy as jnp
from jax import lax
from jax.experimental import pallas as pl
from jax.experimental.pallas import tpu as pltpu
from jax.experimental.pallas import tpu_sc as plsc

NC = 2
NS = 16
NW = NC * NS


def _sc_gather(idx2, table, B, D, b_per_w):
    mesh = plsc.VectorSubcoreMesh(core_axis_name="c", subcore_axis_name="s")

    @functools.partial(
        pl.kernel,
        mesh=mesh,
        out_type=jax.ShapeDtypeStruct((B, D), jnp.float32),
        scratch_types=[
            pltpu.VMEM((b_per_w,), jnp.int32),
            pltpu.VMEM((b_per_w, D), jnp.float32),
            pltpu.SemaphoreType.DMA,
            pltpu.SemaphoreType.DMA,
        ],
    )
    def k(idx_hbm, table_hbm, out_hbm, idx_v, rows_v, gsem, ssem):
        wid = lax.axis_index("s") * NC + lax.axis_index("c")
        base = wid * b_per_w
        pltpu.sync_copy(idx_hbm.at[wid], idx_v)

        def body(g, _):
            v = idx_v[pl.ds(g * 16, 16)]
            for j in range(16):
                r = lax.squeeze(lax.slice_in_dim(v, j, j + 1), (0,))
                pltpu.async_copy(
                    table_hbm.at[pl.ds(r, 1)],
                    rows_v.at[pl.ds(g * 16 + j, 1)],
                    gsem,
                )
            return ()

        lax.fori_loop(0, b_per_w // 16, body, ())
        pltpu.make_async_copy(
            table_hbm.at[pl.ds(0, b_per_w)], rows_v, gsem
        ).wait()
        pltpu.async_copy(rows_v, out_hbm.at[pl.ds(base, b_per_w)], ssem).wait()

    return k(idx2, table)


def kernel(drug_ids, table):
    B, = drug_ids.shape
    _, D = table.shape
    b_per_w = B // NW
    idx2 = drug_ids.astype(jnp.int32).reshape(NW, b_per_w)
    return _sc_gather(idx2, table, B, D, b_per_w)
